# Initial kernel scaffold; baseline (speedup 1.0000x reference)
import jax
import jax.numpy as jnp
from jax.experimental import pallas as pl


def kernel(hidden_states, weight):
    raise NotImplementedError("write your pallas kernel here")


# speedup vs baseline: 7.9027x; 7.9027x over previous
import functools

import jax
import jax.numpy as jnp
from jax.experimental import pallas as pl

NUM_TOKENS = 16384
HIDDEN = 4096
NUM_EXPERTS = 64
TOP_K = 8
TOKEN_BLOCK = 512


def _router_kernel(h_ref, w_ref, merge_ref, map_ref, counts_ref, logits_ref):
    i = pl.program_id(0)

    logits = jax.lax.dot_general(
        h_ref[...], w_ref[...],
        (((1,), (1,)), ((), ())),
        preferred_element_type=jnp.float32,
        precision=jax.lax.Precision.DEFAULT,
    )
    logits_ref[...] = logits

    m = jnp.max(logits, axis=1, keepdims=True)
    e = jnp.exp(logits - m)
    probs = e / jnp.sum(e, axis=1, keepdims=True)

    iota = jax.lax.broadcasted_iota(jnp.int32, probs.shape, 1)
    running = probs
    mask = jnp.zeros(probs.shape, dtype=jnp.bool_)
    ksum = jnp.zeros((probs.shape[0], 1), dtype=jnp.float32)
    for _ in range(TOP_K):
        cur = jnp.max(running, axis=1, keepdims=True)
        eq = running == cur
        first = jnp.min(jnp.where(eq, iota, NUM_EXPERTS), axis=1, keepdims=True)
        sel = iota == first
        mask = mask | sel
        ksum = ksum + cur
        running = jnp.where(sel, -1.0, running)

    merge_ref[...] = jnp.where(mask, probs / ksum, 0.0)
    map_i32 = mask.astype(jnp.int32)
    map_ref[...] = map_i32

    part = jnp.sum(map_i32, axis=0)

    @pl.when(i == 0)
    def _init():
        counts_ref[...] = jnp.zeros_like(counts_ref)

    counts_ref[...] += part


@functools.partial(jax.jit, static_argnames=())
def kernel(hidden_states, weight):
    num_tokens = hidden_states.shape[0]
    grid = (num_tokens // TOKEN_BLOCK,)
    out = pl.pallas_call(
        _router_kernel,
        grid=grid,
        in_specs=[
            pl.BlockSpec((TOKEN_BLOCK, HIDDEN), lambda i: (i, 0)),
            pl.BlockSpec((NUM_EXPERTS, HIDDEN), lambda i: (0, 0)),
        ],
        out_specs=[
            pl.BlockSpec((TOKEN_BLOCK, NUM_EXPERTS), lambda i: (i, 0)),
            pl.BlockSpec((TOKEN_BLOCK, NUM_EXPERTS), lambda i: (i, 0)),
            pl.BlockSpec((NUM_EXPERTS,), lambda i: (0,)),
            pl.BlockSpec((TOKEN_BLOCK, NUM_EXPERTS), lambda i: (i, 0)),
        ],
        out_shape=[
            jax.ShapeDtypeStruct((num_tokens, NUM_EXPERTS), jnp.float32),
            jax.ShapeDtypeStruct((num_tokens, NUM_EXPERTS), jnp.int32),
            jax.ShapeDtypeStruct((NUM_EXPERTS,), jnp.int32),
            jax.ShapeDtypeStruct((num_tokens, NUM_EXPERTS), jnp.float32),
        ],
    )(hidden_states, weight)
    merging_probs, routing_map, tokens_per_expert, router_logits = out
    return merging_probs, routing_map, tokens_per_expert, router_logits


# speedup vs baseline: 8.7222x; 1.1037x over previous
import functools

import jax
import jax.numpy as jnp
from jax.experimental import pallas as pl

NUM_TOKENS = 16384
HIDDEN = 4096
NUM_EXPERTS = 64
TOP_K = 8
TOKEN_BLOCK = 512


def _router_kernel(h_ref, w_ref, merge_ref, map_ref, counts_ref, logits_ref):
    i = pl.program_id(0)

    logits = jax.lax.dot_general(
        h_ref[...], w_ref[...],
        (((1,), (1,)), ((), ())),
        preferred_element_type=jnp.float32,
        precision=jax.lax.Precision.DEFAULT,
    )
    logits_ref[...] = logits

    m = jnp.max(logits, axis=1, keepdims=True)
    e = jnp.exp(logits - m)
    probs = e / jnp.sum(e, axis=1, keepdims=True)

    fiota = jax.lax.broadcasted_iota(jnp.int32, probs.shape, 1).astype(jnp.float32)
    running = probs
    mask = jnp.zeros(probs.shape, dtype=jnp.bool_)
    ksum = jnp.zeros((probs.shape[0], 1), dtype=jnp.float32)
    for _ in range(TOP_K):
        cur = jnp.max(running, axis=1, keepdims=True)
        eq = running == cur
        first = jnp.min(jnp.where(eq, fiota, jnp.float32(NUM_EXPERTS)),
                        axis=1, keepdims=True)
        sel = fiota == first
        mask = mask | sel
        ksum = ksum + cur
        running = jnp.where(sel, -1.0, running)

    merge_ref[...] = jnp.where(mask, probs / ksum, 0.0)
    map_i32 = mask.astype(jnp.int32)
    map_ref[...] = map_i32

    part = jnp.sum(map_i32, axis=0)

    @pl.when(i == 0)
    def _init():
        counts_ref[...] = jnp.zeros_like(counts_ref)

    counts_ref[...] += part


@functools.partial(jax.jit, static_argnames=())
def kernel(hidden_states, weight):
    num_tokens = hidden_states.shape[0]
    grid = (num_tokens // TOKEN_BLOCK,)
    out = pl.pallas_call(
        _router_kernel,
        grid=grid,
        in_specs=[
            pl.BlockSpec((TOKEN_BLOCK, HIDDEN), lambda i: (i, 0)),
            pl.BlockSpec((NUM_EXPERTS, HIDDEN), lambda i: (0, 0)),
        ],
        out_specs=[
            pl.BlockSpec((TOKEN_BLOCK, NUM_EXPERTS), lambda i: (i, 0)),
            pl.BlockSpec((TOKEN_BLOCK, NUM_EXPERTS), lambda i: (i, 0)),
            pl.BlockSpec((NUM_EXPERTS,), lambda i: (0,)),
            pl.BlockSpec((TOKEN_BLOCK, NUM_EXPERTS), lambda i: (i, 0)),
        ],
        out_shape=[
            jax.ShapeDtypeStruct((num_tokens, NUM_EXPERTS), jnp.float32),
            jax.ShapeDtypeStruct((num_tokens, NUM_EXPERTS), jnp.int32),
            jax.ShapeDtypeStruct((NUM_EXPERTS,), jnp.int32),
            jax.ShapeDtypeStruct((num_tokens, NUM_EXPERTS), jnp.float32),
        ],
    )(hidden_states, weight)
    merging_probs, routing_map, tokens_per_expert, router_logits = out
    return merging_probs, routing_map, tokens_per_expert, router_logits


# speedup vs baseline: 9.6500x; 1.1064x over previous
import functools

import jax
import jax.numpy as jnp
from jax.experimental import pallas as pl

NUM_TOKENS = 16384
HIDDEN = 4096
NUM_EXPERTS = 64
TOP_K = 8
TOKEN_BLOCK = 1024


def _router_kernel(h_ref, w_ref, merge_ref, map_ref, counts_ref, logits_ref):
    i = pl.program_id(0)

    logits = jax.lax.dot_general(
        h_ref[...], w_ref[...],
        (((1,), (1,)), ((), ())),
        preferred_element_type=jnp.float32,
        precision=jax.lax.Precision.DEFAULT,
    )
    logits_ref[...] = logits

    m = jnp.max(logits, axis=1, keepdims=True)
    e = jnp.exp(logits - m)
    probs = e / jnp.sum(e, axis=1, keepdims=True)

    fiota = jax.lax.broadcasted_iota(jnp.int32, probs.shape, 1).astype(jnp.float32)
    running = probs
    mask = jnp.zeros(probs.shape, dtype=jnp.bool_)
    ksum = jnp.zeros((probs.shape[0], 1), dtype=jnp.float32)
    for _ in range(TOP_K):
        cur = jnp.max(running, axis=1, keepdims=True)
        eq = running == cur
        first = jnp.min(jnp.where(eq, fiota, jnp.float32(NUM_EXPERTS)),
                        axis=1, keepdims=True)
        sel = fiota == first
        mask = mask | sel
        ksum = ksum + cur
        running = jnp.where(sel, -1.0, running)

    merge_ref[...] = jnp.where(mask, probs / ksum, 0.0)
    map_i32 = mask.astype(jnp.int32)
    map_ref[...] = map_i32

    part = jnp.sum(map_i32, axis=0)

    @pl.when(i == 0)
    def _init():
        counts_ref[...] = jnp.zeros_like(counts_ref)

    counts_ref[...] += part


@functools.partial(jax.jit, static_argnames=())
def kernel(hidden_states, weight):
    num_tokens = hidden_states.shape[0]
    grid = (num_tokens // TOKEN_BLOCK,)
    out = pl.pallas_call(
        _router_kernel,
        grid=grid,
        in_specs=[
            pl.BlockSpec((TOKEN_BLOCK, HIDDEN), lambda i: (i, 0)),
            pl.BlockSpec((NUM_EXPERTS, HIDDEN), lambda i: (0, 0)),
        ],
        out_specs=[
            pl.BlockSpec((TOKEN_BLOCK, NUM_EXPERTS), lambda i: (i, 0)),
            pl.BlockSpec((TOKEN_BLOCK, NUM_EXPERTS), lambda i: (i, 0)),
            pl.BlockSpec((NUM_EXPERTS,), lambda i: (0,)),
            pl.BlockSpec((TOKEN_BLOCK, NUM_EXPERTS), lambda i: (i, 0)),
        ],
        out_shape=[
            jax.ShapeDtypeStruct((num_tokens, NUM_EXPERTS), jnp.float32),
            jax.ShapeDtypeStruct((num_tokens, NUM_EXPERTS), jnp.int32),
            jax.ShapeDtypeStruct((NUM_EXPERTS,), jnp.int32),
            jax.ShapeDtypeStruct((num_tokens, NUM_EXPERTS), jnp.float32),
        ],
    )(hidden_states, weight)
    merging_probs, routing_map, tokens_per_expert, router_logits = out
    return merging_probs, routing_map, tokens_per_expert, router_logits


# speedup vs baseline: 9.6698x; 1.0021x over previous
import functools

import jax
import jax.numpy as jnp
from jax.experimental import pallas as pl

NUM_TOKENS = 16384
HIDDEN = 4096
NUM_EXPERTS = 64
TOP_K = 8
TOKEN_BLOCK = 1024


def _router_kernel(h0_ref, h1_ref, w_ref, merge_ref, map_ref, counts_ref,
                   logits_ref):
    i = pl.program_id(0)

    half = HIDDEN // 2
    logits = jax.lax.dot_general(
        h0_ref[...], w_ref[:, :half],
        (((1,), (1,)), ((), ())),
        preferred_element_type=jnp.float32,
        precision=jax.lax.Precision.DEFAULT,
    ) + jax.lax.dot_general(
        h1_ref[...], w_ref[:, half:],
        (((1,), (1,)), ((), ())),
        preferred_element_type=jnp.float32,
        precision=jax.lax.Precision.DEFAULT,
    )
    logits_ref[...] = logits

    m = jnp.max(logits, axis=1, keepdims=True)
    e = jnp.exp(logits - m)
    probs = e / jnp.sum(e, axis=1, keepdims=True)

    fiota = jax.lax.broadcasted_iota(jnp.int32, probs.shape, 1).astype(jnp.float32)
    running = probs
    mask = jnp.zeros(probs.shape, dtype=jnp.bool_)
    ksum = jnp.zeros((probs.shape[0], 1), dtype=jnp.float32)
    for _ in range(TOP_K):
        cur = jnp.max(running, axis=1, keepdims=True)
        eq = running == cur
        first = jnp.min(jnp.where(eq, fiota, jnp.float32(NUM_EXPERTS)),
                        axis=1, keepdims=True)
        sel = fiota == first
        mask = mask | sel
        ksum = ksum + cur
        running = jnp.where(sel, -1.0, running)

    merge_ref[...] = jnp.where(mask, probs / ksum, 0.0)
    map_i32 = mask.astype(jnp.int32)
    map_ref[...] = map_i32

    part = jnp.sum(map_i32, axis=0)

    @pl.when(i == 0)
    def _init():
        counts_ref[...] = jnp.zeros_like(counts_ref)

    counts_ref[...] += part


@functools.partial(jax.jit, static_argnames=())
def kernel(hidden_states, weight):
    num_tokens = hidden_states.shape[0]
    grid = (num_tokens // TOKEN_BLOCK,)
    out = pl.pallas_call(
        _router_kernel,
        grid=grid,
        in_specs=[
            pl.BlockSpec((TOKEN_BLOCK, HIDDEN // 2), lambda i: (i, 0)),
            pl.BlockSpec((TOKEN_BLOCK, HIDDEN // 2), lambda i: (i, 1)),
            pl.BlockSpec((NUM_EXPERTS, HIDDEN), lambda i: (0, 0)),
        ],
        out_specs=[
            pl.BlockSpec((TOKEN_BLOCK, NUM_EXPERTS), lambda i: (i, 0)),
            pl.BlockSpec((TOKEN_BLOCK, NUM_EXPERTS), lambda i: (i, 0)),
            pl.BlockSpec((NUM_EXPERTS,), lambda i: (0,)),
            pl.BlockSpec((TOKEN_BLOCK, NUM_EXPERTS), lambda i: (i, 0)),
        ],
        out_shape=[
            jax.ShapeDtypeStruct((num_tokens, NUM_EXPERTS), jnp.float32),
            jax.ShapeDtypeStruct((num_tokens, NUM_EXPERTS), jnp.int32),
            jax.ShapeDtypeStruct((NUM_EXPERTS,), jnp.int32),
            jax.ShapeDtypeStruct((num_tokens, NUM_EXPERTS), jnp.float32),
        ],
    )(hidden_states, hidden_states, weight)
    merging_probs, routing_map, tokens_per_expert, router_logits = out
    return merging_probs, routing_map, tokens_per_expert, router_logits


# speedup vs baseline: 10.1094x; 1.0455x over previous
import functools

import jax
import jax.numpy as jnp
from jax.experimental import pallas as pl

NUM_TOKENS = 16384
HIDDEN = 4096
NUM_EXPERTS = 64
TOP_K = 8
TOKEN_BLOCK = 1024


def _router_kernel(h0_ref, h1_ref, w_ref, merge_ref, map_ref, counts_ref,
                   logits_ref):
    i = pl.program_id(0)

    half = HIDDEN // 2
    logits = jax.lax.dot_general(
        h0_ref[...], w_ref[:, :half],
        (((1,), (1,)), ((), ())),
        preferred_element_type=jnp.float32,
        precision=jax.lax.Precision.DEFAULT,
    ) + jax.lax.dot_general(
        h1_ref[...], w_ref[:, half:],
        (((1,), (1,)), ((), ())),
        preferred_element_type=jnp.float32,
        precision=jax.lax.Precision.DEFAULT,
    )
    logits_ref[...] = logits

    e = jnp.exp(logits)
    probs = e / jnp.sum(e, axis=1, keepdims=True)

    iota = jax.lax.broadcasted_iota(jnp.int32, probs.shape, 1)
    running = probs
    mask = jnp.zeros(probs.shape, dtype=jnp.bool_)
    for _ in range(TOP_K):
        idx = jnp.argmax(running, axis=1, keepdims=True)
        sel = iota == idx
        mask = mask | sel
        running = jnp.where(sel, -1.0, running)

    ksum = jnp.sum(jnp.where(mask, probs, 0.0), axis=1, keepdims=True)
    merge_ref[...] = jnp.where(mask, probs / ksum, 0.0)
    map_i32 = mask.astype(jnp.int32)
    map_ref[...] = map_i32

    part = jnp.sum(map_i32, axis=0)

    @pl.when(i == 0)
    def _init():
        counts_ref[...] = jnp.zeros_like(counts_ref)

    counts_ref[...] += part


@functools.partial(jax.jit, static_argnames=())
def kernel(hidden_states, weight):
    num_tokens = hidden_states.shape[0]
    grid = (num_tokens // TOKEN_BLOCK,)
    out = pl.pallas_call(
        _router_kernel,
        grid=grid,
        in_specs=[
            pl.BlockSpec((TOKEN_BLOCK, HIDDEN // 2), lambda i: (i, 0)),
            pl.BlockSpec((TOKEN_BLOCK, HIDDEN // 2), lambda i: (i, 1)),
            pl.BlockSpec((NUM_EXPERTS, HIDDEN), lambda i: (0, 0)),
        ],
        out_specs=[
            pl.BlockSpec((TOKEN_BLOCK, NUM_EXPERTS), lambda i: (i, 0)),
            pl.BlockSpec((TOKEN_BLOCK, NUM_EXPERTS), lambda i: (i, 0)),
            pl.BlockSpec((NUM_EXPERTS,), lambda i: (0,)),
            pl.BlockSpec((TOKEN_BLOCK, NUM_EXPERTS), lambda i: (i, 0)),
        ],
        out_shape=[
            jax.ShapeDtypeStruct((num_tokens, NUM_EXPERTS), jnp.float32),
            jax.ShapeDtypeStruct((num_tokens, NUM_EXPERTS), jnp.int32),
            jax.ShapeDtypeStruct((NUM_EXPERTS,), jnp.int32),
            jax.ShapeDtypeStruct((num_tokens, NUM_EXPERTS), jnp.float32),
        ],
    )(hidden_states, hidden_states, weight)
    merging_probs, routing_map, tokens_per_expert, router_logits = out
    return merging_probs, routing_map, tokens_per_expert, router_logits
